# baseline (speedup 1.0000x reference)
import functools

import jax
import jax.numpy as jnp
from jax import lax
from jax.experimental import pallas as pl
from jax.experimental.pallas import tpu as pltpu
from jax.experimental.pallas import tpu_sc as plsc

_NC, _NS = 2, 16
_NW = _NC * _NS
_CH = 128
_SPB = 1
_BC = _CH * _SPB
_TBLK = 512


def _tc_widen(t_t):
    d, v = t_t.shape

    def body(in_ref, out_ref):
        xt = in_ref[...].T
        out_ref[:, 0:d] = xt
        out_ref[:, d:2 * d] = xt

    return pl.pallas_call(
        body,
        grid=(pl.cdiv(v, _TBLK),),
        in_specs=[pl.BlockSpec((d, _TBLK), lambda i: (0, i))],
        out_specs=pl.BlockSpec((_TBLK, 2 * d), lambda i: (i, 0)),
        out_shape=jax.ShapeDtypeStruct((v, 2 * d), jnp.float32),
    )(t_t)


def _make_gather(n_idx: int, d2: int):
    per_w = n_idx // _NW
    nbc = per_w // _BC
    assert n_idx == _NW * nbc * _BC and nbc % 2 == 0 and nbc >= 4

    mesh = plsc.VectorSubcoreMesh(core_axis_name="c", subcore_axis_name="s")

    @functools.partial(
        pl.kernel,
        out_type=jax.ShapeDtypeStruct((n_idx, d2), jnp.float32),
        mesh=mesh,
        scratch_types=[
            pltpu.VMEM((per_w,), jnp.int32),
            pltpu.VMEM((_BC, d2), jnp.float32),
            pltpu.VMEM((_BC, d2), jnp.float32),
            pltpu.SemaphoreType.DMA,
            pltpu.SemaphoreType.DMA,
            pltpu.SemaphoreType.DMA,
            pltpu.SemaphoreType.DMA,
        ],
        compiler_params=pltpu.CompilerParams(use_tc_tiling_on_sc=True),
    )
    def gather(table_hbm, idx_hbm, out_hbm, idx_v,
               buf_a, buf_b, gsem_a, gsem_b, wsem_a, wsem_b):
        wid = lax.axis_index("s") * _NC + lax.axis_index("c")
        pltpu.sync_copy(idx_hbm.at[pl.ds(wid * per_w, per_w)], idx_v)
        row0 = wid * per_w

        def g_start(c, buf, sem):
            for k in range(_SPB):
                pltpu.make_async_copy(
                    table_hbm.at[idx_v.at[pl.ds((c * _SPB + k) * _CH, _CH)]],
                    buf.at[pl.ds(k * _CH, _CH)], sem).start()

        def g_wait(buf, sem):
            pltpu.make_async_copy(
                table_hbm.at[pl.ds(0, _BC)], buf, sem).wait()

        def w_start(c, buf, sem):
            pltpu.make_async_copy(
                buf, out_hbm.at[pl.ds(row0 + c * _BC, _BC)], sem).start()

        def w_wait(buf, sem):
            pltpu.make_async_copy(
                buf, out_hbm.at[pl.ds(row0, _BC)], sem).wait()

        g_start(0, buf_a, gsem_a)
        g_start(1, buf_b, gsem_b)

        def step(i, carry):
            c = 2 * i
            g_wait(buf_a, gsem_a)
            w_start(c, buf_a, wsem_a)
            g_wait(buf_b, gsem_b)
            w_start(c + 1, buf_b, wsem_b)
            w_wait(buf_a, wsem_a)
            g_start(c + 2, buf_a, gsem_a)
            w_wait(buf_b, wsem_b)
            g_start(c + 3, buf_b, gsem_b)
            return carry

        lax.fori_loop(0, nbc // 2 - 1, step, 0)

        c = nbc - 2
        g_wait(buf_a, gsem_a)
        w_start(c, buf_a, wsem_a)
        g_wait(buf_b, gsem_b)
        w_start(c + 1, buf_b, wsem_b)
        w_wait(buf_a, wsem_a)
        w_wait(buf_b, wsem_b)

    return gather


def kernel(input_tensor, indices):
    n_idx = indices.size
    d = input_tensor.shape[-1]
    idx = indices.astype(jnp.int32).reshape(-1)
    wide = _tc_widen(input_tensor.T)
    out2 = _make_gather(n_idx, 2 * d)(wide, idx)
    return out2[:, :d].reshape(indices.shape + (d,))


# speedup vs baseline: 1.5427x; 1.5427x over previous
import functools

import jax
import jax.numpy as jnp
from jax import lax
from jax.experimental import pallas as pl
from jax.experimental.pallas import tpu as pltpu
from jax.experimental.pallas import tpu_sc as plsc

_NC, _NS = 2, 16
_NW = _NC * _NS
_CH = 128
_SPB = 1
_BC = _CH * _SPB
_TBLK = 512


def _tc_widen(t_t):
    d, v = t_t.shape

    def body(in_ref, out_ref):
        xt = in_ref[...].T
        out_ref[:, 0:d] = xt
        out_ref[:, d:2 * d] = xt

    return pl.pallas_call(
        body,
        grid=(pl.cdiv(v, _TBLK),),
        in_specs=[pl.BlockSpec((d, _TBLK), lambda i: (0, i))],
        out_specs=pl.BlockSpec((_TBLK, 2 * d), lambda i: (i, 0)),
        out_shape=jax.ShapeDtypeStruct((v, 2 * d), jnp.float32),
    )(t_t)


def _make_gather(n_idx: int, d2: int):
    per_w = n_idx // _NW
    nbc = per_w // _BC
    assert n_idx == _NW * nbc * _BC and nbc % 2 == 0 and nbc >= 4

    mesh = plsc.VectorSubcoreMesh(core_axis_name="c", subcore_axis_name="s")

    @functools.partial(
        pl.kernel,
        out_type=jax.ShapeDtypeStruct((n_idx, d2), jnp.float32),
        mesh=mesh,
        scratch_types=[
            pltpu.VMEM((per_w,), jnp.int32),
            pltpu.VMEM((_BC, d2), jnp.float32),
            pltpu.VMEM((_BC, d2), jnp.float32),
            pltpu.SemaphoreType.DMA,
            pltpu.SemaphoreType.DMA,
            pltpu.SemaphoreType.DMA,
            pltpu.SemaphoreType.DMA,
        ],
        compiler_params=pltpu.CompilerParams(use_tc_tiling_on_sc=True),
    )
    def gather(table_hbm, idx_hbm, out_hbm, idx_v,
               buf_a, buf_b, gsem_a, gsem_b, wsem_a, wsem_b):
        wid = lax.axis_index("s") * _NC + lax.axis_index("c")
        pltpu.sync_copy(idx_hbm.at[pl.ds(wid * per_w, per_w)], idx_v)
        row0 = wid * per_w

        def g_start(c, buf, sem):
            for k in range(_SPB):
                pltpu.make_async_copy(
                    table_hbm.at[idx_v.at[pl.ds((c * _SPB + k) * _CH, _CH)]],
                    buf.at[pl.ds(k * _CH, _CH)], sem).start()

        def g_wait(buf, sem):
            pltpu.make_async_copy(
                table_hbm.at[pl.ds(0, _BC)], buf, sem).wait()

        def w_start(c, buf, sem):
            pltpu.make_async_copy(
                buf, out_hbm.at[pl.ds(row0 + c * _BC, _BC)], sem).start()

        def w_wait(buf, sem):
            pltpu.make_async_copy(
                buf, out_hbm.at[pl.ds(row0, _BC)], sem).wait()

        g_start(0, buf_a, gsem_a)
        g_start(1, buf_b, gsem_b)

        def step(i, carry):
            c = 2 * i
            g_wait(buf_a, gsem_a)
            w_start(c, buf_a, wsem_a)
            g_wait(buf_b, gsem_b)
            w_start(c + 1, buf_b, wsem_b)
            w_wait(buf_a, wsem_a)
            g_start(c + 2, buf_a, gsem_a)
            w_wait(buf_b, wsem_b)
            g_start(c + 3, buf_b, gsem_b)
            return carry

        lax.fori_loop(0, nbc // 2 - 1, step, 0)

        c = nbc - 2
        g_wait(buf_a, gsem_a)
        w_start(c, buf_a, wsem_a)
        g_wait(buf_b, gsem_b)
        w_start(c + 1, buf_b, wsem_b)
        w_wait(buf_a, wsem_a)
        w_wait(buf_b, wsem_b)

    return gather


def kernel(input_tensor, indices):
    n_idx = indices.size
    d = input_tensor.shape[-1]
    idx = indices.astype(jnp.int32).reshape(-1)
    packed = input_tensor.reshape(input_tensor.shape[0] // 2, 2 * d)
    out2 = _make_gather(n_idx, 2 * d)(packed, idx >> 1)
    par = (idx & 1)[:, None] == 1
    out = jnp.where(par, out2[:, d:], out2[:, :d])
    return out.reshape(indices.shape + (d,))


# speedup vs baseline: 1.9702x; 1.2771x over previous
import functools

import jax
import jax.numpy as jnp
from jax import lax
from jax.experimental import pallas as pl
from jax.experimental.pallas import tpu as pltpu
from jax.experimental.pallas import tpu_sc as plsc

_NC, _NS = 2, 16
_NW = _NC * _NS
_CH = 128
_SPB = 1
_BC = _CH * _SPB
_TBLK = 2048


def _tc_widen(t_t):
    d, v = t_t.shape

    def body(in_ref, out_ref):
        xt = in_ref[...].T
        out_ref[:, 0:d] = xt
        out_ref[:, d:2 * d] = xt

    return pl.pallas_call(
        body,
        grid=(pl.cdiv(v, _TBLK),),
        in_specs=[pl.BlockSpec((d, _TBLK), lambda i: (0, i))],
        out_specs=pl.BlockSpec((_TBLK, 2 * d), lambda i: (i, 0)),
        out_shape=jax.ShapeDtypeStruct((v, 2 * d), jnp.float32),
    )(t_t)


def _make_gather(n_idx: int, d2: int):
    per_w = n_idx // _NW
    nbc = per_w // _BC
    assert n_idx == _NW * nbc * _BC and nbc % 2 == 0 and nbc >= 4

    mesh = plsc.VectorSubcoreMesh(core_axis_name="c", subcore_axis_name="s")

    @functools.partial(
        pl.kernel,
        out_type=jax.ShapeDtypeStruct((n_idx, d2), jnp.float32),
        mesh=mesh,
        scratch_types=[
            pltpu.VMEM((per_w,), jnp.int32),
            pltpu.VMEM((_BC, d2), jnp.float32),
            pltpu.VMEM((_BC, d2), jnp.float32),
            pltpu.SemaphoreType.DMA,
            pltpu.SemaphoreType.DMA,
            pltpu.SemaphoreType.DMA,
            pltpu.SemaphoreType.DMA,
        ],
        compiler_params=pltpu.CompilerParams(use_tc_tiling_on_sc=True),
    )
    def gather(table_hbm, idx_hbm, out_hbm, idx_v,
               buf_a, buf_b, gsem_a, gsem_b, wsem_a, wsem_b):
        wid = lax.axis_index("s") * _NC + lax.axis_index("c")
        pltpu.sync_copy(idx_hbm.at[pl.ds(wid * per_w, per_w)], idx_v)
        row0 = wid * per_w

        def g_start(c, buf, sem):
            for k in range(_SPB):
                pltpu.make_async_copy(
                    table_hbm.at[idx_v.at[pl.ds((c * _SPB + k) * _CH, _CH)]],
                    buf.at[pl.ds(k * _CH, _CH)], sem).start()

        def g_wait(buf, sem):
            pltpu.make_async_copy(
                table_hbm.at[pl.ds(0, _BC)], buf, sem).wait()

        def w_start(c, buf, sem):
            pltpu.make_async_copy(
                buf, out_hbm.at[pl.ds(row0 + c * _BC, _BC)], sem).start()

        def w_wait(buf, sem):
            pltpu.make_async_copy(
                buf, out_hbm.at[pl.ds(row0, _BC)], sem).wait()

        g_start(0, buf_a, gsem_a)
        g_start(1, buf_b, gsem_b)

        def step(i, carry):
            c = 2 * i
            g_wait(buf_a, gsem_a)
            w_start(c, buf_a, wsem_a)
            g_wait(buf_b, gsem_b)
            w_start(c + 1, buf_b, wsem_b)
            w_wait(buf_a, wsem_a)
            g_start(c + 2, buf_a, gsem_a)
            w_wait(buf_b, wsem_b)
            g_start(c + 3, buf_b, gsem_b)
            return carry

        lax.fori_loop(0, nbc // 2 - 1, step, 0)

        c = nbc - 2
        g_wait(buf_a, gsem_a)
        w_start(c, buf_a, wsem_a)
        g_wait(buf_b, gsem_b)
        w_start(c + 1, buf_b, wsem_b)
        w_wait(buf_a, wsem_a)
        w_wait(buf_b, wsem_b)

    return gather


def kernel(input_tensor, indices):
    n_idx = indices.size
    d = input_tensor.shape[-1]
    idx = indices.astype(jnp.int32).reshape(-1)
    wide = _tc_widen(input_tensor.T)
    out2 = _make_gather(n_idx, 2 * d)(wide, idx)
    return out2[:, :d].reshape(indices.shape + (d,))


# speedup vs baseline: 2.6378x; 1.3388x over previous
import functools

import jax
import jax.numpy as jnp
from jax import lax
from jax.experimental import pallas as pl
from jax.experimental.pallas import tpu as pltpu
from jax.experimental.pallas import tpu_sc as plsc

_NC, _NS = 2, 16
_NW = _NC * _NS
_CH = 128
_SPB = 1
_BC = _CH * _SPB
_TBLK = 8192


def _tc_widen(t_t):
    d, v = t_t.shape

    def body(in_ref, out_ref):
        xt = in_ref[...].T
        out_ref[:, 0:d] = xt
        out_ref[:, d:2 * d] = xt

    return pl.pallas_call(
        body,
        grid=(pl.cdiv(v, _TBLK),),
        in_specs=[pl.BlockSpec((d, _TBLK), lambda i: (0, i))],
        out_specs=pl.BlockSpec((_TBLK, 2 * d), lambda i: (i, 0)),
        out_shape=jax.ShapeDtypeStruct((v, 2 * d), jnp.float32),
    )(t_t)


def _make_gather(n_idx: int, d2: int):
    per_w = n_idx // _NW
    nbc = per_w // _BC
    assert n_idx == _NW * nbc * _BC and nbc % 2 == 0 and nbc >= 4

    mesh = plsc.VectorSubcoreMesh(core_axis_name="c", subcore_axis_name="s")

    @functools.partial(
        pl.kernel,
        out_type=jax.ShapeDtypeStruct((n_idx, d2), jnp.float32),
        mesh=mesh,
        scratch_types=[
            pltpu.VMEM((per_w,), jnp.int32),
            pltpu.VMEM((_BC, d2), jnp.float32),
            pltpu.VMEM((_BC, d2), jnp.float32),
            pltpu.SemaphoreType.DMA,
            pltpu.SemaphoreType.DMA,
            pltpu.SemaphoreType.DMA,
            pltpu.SemaphoreType.DMA,
        ],
        compiler_params=pltpu.CompilerParams(use_tc_tiling_on_sc=True),
    )
    def gather(table_hbm, idx_hbm, out_hbm, idx_v,
               buf_a, buf_b, gsem_a, gsem_b, wsem_a, wsem_b):
        wid = lax.axis_index("s") * _NC + lax.axis_index("c")
        pltpu.sync_copy(idx_hbm.at[pl.ds(wid * per_w, per_w)], idx_v)
        row0 = wid * per_w

        def g_start(c, buf, sem):
            for k in range(_SPB):
                pltpu.make_async_copy(
                    table_hbm.at[idx_v.at[pl.ds((c * _SPB + k) * _CH, _CH)]],
                    buf.at[pl.ds(k * _CH, _CH)], sem).start()

        def g_wait(buf, sem):
            pltpu.make_async_copy(
                table_hbm.at[pl.ds(0, _BC)], buf, sem).wait()

        def w_start(c, buf, sem):
            pltpu.make_async_copy(
                buf, out_hbm.at[pl.ds(row0 + c * _BC, _BC)], sem).start()

        def w_wait(buf, sem):
            pltpu.make_async_copy(
                buf, out_hbm.at[pl.ds(row0, _BC)], sem).wait()

        g_start(0, buf_a, gsem_a)
        g_start(1, buf_b, gsem_b)

        def step(i, carry):
            c = 2 * i
            g_wait(buf_a, gsem_a)
            w_start(c, buf_a, wsem_a)
            g_wait(buf_b, gsem_b)
            w_start(c + 1, buf_b, wsem_b)
            w_wait(buf_a, wsem_a)
            g_start(c + 2, buf_a, gsem_a)
            w_wait(buf_b, wsem_b)
            g_start(c + 3, buf_b, gsem_b)
            return carry

        lax.fori_loop(0, nbc // 2 - 1, step, 0)

        c = nbc - 2
        g_wait(buf_a, gsem_a)
        w_start(c, buf_a, wsem_a)
        g_wait(buf_b, gsem_b)
        w_start(c + 1, buf_b, wsem_b)
        w_wait(buf_a, wsem_a)
        w_wait(buf_b, wsem_b)

    return gather


def kernel(input_tensor, indices):
    n_idx = indices.size
    d = input_tensor.shape[-1]
    idx = indices.astype(jnp.int32).reshape(-1)
    wide = _tc_widen(input_tensor.T)
    out2 = _make_gather(n_idx, 2 * d)(wide, idx)
    return out2[:, :d].reshape(indices.shape + (d,))


# speedup vs baseline: 2.8092x; 1.0650x over previous
import functools

import jax
import jax.numpy as jnp
from jax import lax
from jax.experimental import pallas as pl
from jax.experimental.pallas import tpu as pltpu
from jax.experimental.pallas import tpu_sc as plsc

_NC, _NS = 2, 16
_NW = _NC * _NS
_CH = 128
_SPB = 1
_BC = _CH * _SPB
_TBLK = 16384


def _tc_widen(t_t):
    d, v = t_t.shape

    def body(in_ref, out_ref):
        xt = in_ref[...].T
        out_ref[:, 0:d] = xt
        out_ref[:, d:2 * d] = xt

    return pl.pallas_call(
        body,
        grid=(pl.cdiv(v, _TBLK),),
        in_specs=[pl.BlockSpec((d, _TBLK), lambda i: (0, i))],
        out_specs=pl.BlockSpec((_TBLK, 2 * d), lambda i: (i, 0)),
        out_shape=jax.ShapeDtypeStruct((v, 2 * d), jnp.float32),
    )(t_t)


def _make_gather(n_idx: int, d2: int):
    per_w = n_idx // _NW
    nbc = per_w // _BC
    assert n_idx == _NW * nbc * _BC and nbc % 2 == 0 and nbc >= 4

    mesh = plsc.VectorSubcoreMesh(core_axis_name="c", subcore_axis_name="s")

    @functools.partial(
        pl.kernel,
        out_type=jax.ShapeDtypeStruct((n_idx, d2), jnp.float32),
        mesh=mesh,
        scratch_types=[
            pltpu.VMEM((per_w,), jnp.int32),
            pltpu.VMEM((_BC, d2), jnp.float32),
            pltpu.VMEM((_BC, d2), jnp.float32),
            pltpu.SemaphoreType.DMA,
            pltpu.SemaphoreType.DMA,
            pltpu.SemaphoreType.DMA,
            pltpu.SemaphoreType.DMA,
        ],
        compiler_params=pltpu.CompilerParams(use_tc_tiling_on_sc=True),
    )
    def gather(table_hbm, idx_hbm, out_hbm, idx_v,
               buf_a, buf_b, gsem_a, gsem_b, wsem_a, wsem_b):
        wid = lax.axis_index("s") * _NC + lax.axis_index("c")
        pltpu.sync_copy(idx_hbm.at[pl.ds(wid * per_w, per_w)], idx_v)
        row0 = wid * per_w

        def g_start(c, buf, sem):
            for k in range(_SPB):
                pltpu.make_async_copy(
                    table_hbm.at[idx_v.at[pl.ds((c * _SPB + k) * _CH, _CH)]],
                    buf.at[pl.ds(k * _CH, _CH)], sem).start()

        def g_wait(buf, sem):
            pltpu.make_async_copy(
                table_hbm.at[pl.ds(0, _BC)], buf, sem).wait()

        def w_start(c, buf, sem):
            pltpu.make_async_copy(
                buf, out_hbm.at[pl.ds(row0 + c * _BC, _BC)], sem).start()

        def w_wait(buf, sem):
            pltpu.make_async_copy(
                buf, out_hbm.at[pl.ds(row0, _BC)], sem).wait()

        g_start(0, buf_a, gsem_a)
        g_start(1, buf_b, gsem_b)

        def step(i, carry):
            c = 2 * i
            g_wait(buf_a, gsem_a)
            w_start(c, buf_a, wsem_a)
            g_wait(buf_b, gsem_b)
            w_start(c + 1, buf_b, wsem_b)
            w_wait(buf_a, wsem_a)
            g_start(c + 2, buf_a, gsem_a)
            w_wait(buf_b, wsem_b)
            g_start(c + 3, buf_b, gsem_b)
            return carry

        lax.fori_loop(0, nbc // 2 - 1, step, 0)

        c = nbc - 2
        g_wait(buf_a, gsem_a)
        w_start(c, buf_a, wsem_a)
        g_wait(buf_b, gsem_b)
        w_start(c + 1, buf_b, wsem_b)
        w_wait(buf_a, wsem_a)
        w_wait(buf_b, wsem_b)

    return gather


def kernel(input_tensor, indices):
    n_idx = indices.size
    d = input_tensor.shape[-1]
    idx = indices.astype(jnp.int32).reshape(-1)
    wide = _tc_widen(input_tensor.T)
    out2 = _make_gather(n_idx, 2 * d)(wide, idx)
    return out2[:, :d].reshape(indices.shape + (d,))


# speedup vs baseline: 3.0413x; 1.0826x over previous
import functools

import jax
import jax.numpy as jnp
from jax import lax
from jax.experimental import pallas as pl
from jax.experimental.pallas import tpu as pltpu
from jax.experimental.pallas import tpu_sc as plsc

_NC, _NS = 2, 16
_NW = _NC * _NS
_CH = 128
_SPB = 1
_BC = _CH * _SPB
_TBLK = 16384


def _tc_widen(t_t):
    d, v = t_t.shape

    def body(in_ref, out_ref):
        out_ref[:, 0:d] = in_ref[...].T

    return pl.pallas_call(
        body,
        grid=(pl.cdiv(v, _TBLK),),
        in_specs=[pl.BlockSpec((d, _TBLK), lambda i: (0, i))],
        out_specs=pl.BlockSpec((_TBLK, 2 * d), lambda i: (i, 0)),
        out_shape=jax.ShapeDtypeStruct((v, 2 * d), jnp.float32),
    )(t_t)


def _make_gather(n_idx: int, d2: int):
    per_w = n_idx // _NW
    nbc = per_w // _BC
    assert n_idx == _NW * nbc * _BC and nbc % 2 == 0 and nbc >= 4

    mesh = plsc.VectorSubcoreMesh(core_axis_name="c", subcore_axis_name="s")

    @functools.partial(
        pl.kernel,
        out_type=jax.ShapeDtypeStruct((n_idx, d2), jnp.float32),
        mesh=mesh,
        scratch_types=[
            pltpu.VMEM((per_w,), jnp.int32),
            pltpu.VMEM((_BC, d2), jnp.float32),
            pltpu.VMEM((_BC, d2), jnp.float32),
            pltpu.SemaphoreType.DMA,
            pltpu.SemaphoreType.DMA,
            pltpu.SemaphoreType.DMA,
            pltpu.SemaphoreType.DMA,
        ],
        compiler_params=pltpu.CompilerParams(use_tc_tiling_on_sc=True),
    )
    def gather(table_hbm, idx_hbm, out_hbm, idx_v,
               buf_a, buf_b, gsem_a, gsem_b, wsem_a, wsem_b):
        wid = lax.axis_index("s") * _NC + lax.axis_index("c")
        pltpu.sync_copy(idx_hbm.at[pl.ds(wid * per_w, per_w)], idx_v)
        row0 = wid * per_w

        def g_start(c, buf, sem):
            for k in range(_SPB):
                pltpu.make_async_copy(
                    table_hbm.at[idx_v.at[pl.ds((c * _SPB + k) * _CH, _CH)]],
                    buf.at[pl.ds(k * _CH, _CH)], sem).start()

        def g_wait(buf, sem):
            pltpu.make_async_copy(
                table_hbm.at[pl.ds(0, _BC)], buf, sem).wait()

        def w_start(c, buf, sem):
            pltpu.make_async_copy(
                buf, out_hbm.at[pl.ds(row0 + c * _BC, _BC)], sem).start()

        def w_wait(buf, sem):
            pltpu.make_async_copy(
                buf, out_hbm.at[pl.ds(row0, _BC)], sem).wait()

        g_start(0, buf_a, gsem_a)
        g_start(1, buf_b, gsem_b)

        def step(i, carry):
            c = 2 * i
            g_wait(buf_a, gsem_a)
            w_start(c, buf_a, wsem_a)
            g_wait(buf_b, gsem_b)
            w_start(c + 1, buf_b, wsem_b)
            w_wait(buf_a, wsem_a)
            g_start(c + 2, buf_a, gsem_a)
            w_wait(buf_b, wsem_b)
            g_start(c + 3, buf_b, gsem_b)
            return carry

        lax.fori_loop(0, nbc // 2 - 1, step, 0)

        c = nbc - 2
        g_wait(buf_a, gsem_a)
        w_start(c, buf_a, wsem_a)
        g_wait(buf_b, gsem_b)
        w_start(c + 1, buf_b, wsem_b)
        w_wait(buf_a, wsem_a)
        w_wait(buf_b, wsem_b)

    return gather


def kernel(input_tensor, indices):
    n_idx = indices.size
    d = input_tensor.shape[-1]
    idx = indices.astype(jnp.int32).reshape(-1)
    wide = _tc_widen(input_tensor.T)
    out2 = _make_gather(n_idx, 2 * d)(wide, idx)
    return out2[:, :d].reshape(indices.shape + (d,))


# speedup vs baseline: 3.0772x; 1.0118x over previous
import functools

import jax
import jax.numpy as jnp
from jax import lax
from jax.experimental import pallas as pl
from jax.experimental.pallas import tpu as pltpu
from jax.experimental.pallas import tpu_sc as plsc

_NC, _NS = 2, 16
_NW = _NC * _NS
_CH = 128
_SPB = 1
_BC = _CH * _SPB
_TBLK = 32768


def _tc_widen(t_t):
    d, v = t_t.shape

    def body(in_ref, out_ref):
        out_ref[:, 0:d] = in_ref[...].T

    return pl.pallas_call(
        body,
        grid=(pl.cdiv(v, _TBLK),),
        in_specs=[pl.BlockSpec((d, _TBLK), lambda i: (0, i))],
        out_specs=pl.BlockSpec((_TBLK, 2 * d), lambda i: (i, 0)),
        out_shape=jax.ShapeDtypeStruct((v, 2 * d), jnp.float32),
        compiler_params=pltpu.CompilerParams(
            vmem_limit_bytes=100 * 1024 * 1024),
    )(t_t)


def _make_gather(n_idx: int, d2: int):
    per_w = n_idx // _NW
    nbc = per_w // _BC
    assert n_idx == _NW * nbc * _BC and nbc % 2 == 0 and nbc >= 4

    mesh = plsc.VectorSubcoreMesh(core_axis_name="c", subcore_axis_name="s")

    @functools.partial(
        pl.kernel,
        out_type=jax.ShapeDtypeStruct((n_idx, d2), jnp.float32),
        mesh=mesh,
        scratch_types=[
            pltpu.VMEM((per_w,), jnp.int32),
            pltpu.VMEM((_BC, d2), jnp.float32),
            pltpu.VMEM((_BC, d2), jnp.float32),
            pltpu.SemaphoreType.DMA,
            pltpu.SemaphoreType.DMA,
            pltpu.SemaphoreType.DMA,
            pltpu.SemaphoreType.DMA,
        ],
        compiler_params=pltpu.CompilerParams(use_tc_tiling_on_sc=True),
    )
    def gather(table_hbm, idx_hbm, out_hbm, idx_v,
               buf_a, buf_b, gsem_a, gsem_b, wsem_a, wsem_b):
        wid = lax.axis_index("s") * _NC + lax.axis_index("c")
        pltpu.sync_copy(idx_hbm.at[pl.ds(wid * per_w, per_w)], idx_v)
        row0 = wid * per_w

        def g_start(c, buf, sem):
            for k in range(_SPB):
                pltpu.make_async_copy(
                    table_hbm.at[idx_v.at[pl.ds((c * _SPB + k) * _CH, _CH)]],
                    buf.at[pl.ds(k * _CH, _CH)], sem).start()

        def g_wait(buf, sem):
            pltpu.make_async_copy(
                table_hbm.at[pl.ds(0, _BC)], buf, sem).wait()

        def w_start(c, buf, sem):
            pltpu.make_async_copy(
                buf, out_hbm.at[pl.ds(row0 + c * _BC, _BC)], sem).start()

        def w_wait(buf, sem):
            pltpu.make_async_copy(
                buf, out_hbm.at[pl.ds(row0, _BC)], sem).wait()

        g_start(0, buf_a, gsem_a)
        g_start(1, buf_b, gsem_b)

        def step(i, carry):
            c = 2 * i
            g_wait(buf_a, gsem_a)
            w_start(c, buf_a, wsem_a)
            g_wait(buf_b, gsem_b)
            w_start(c + 1, buf_b, wsem_b)
            w_wait(buf_a, wsem_a)
            g_start(c + 2, buf_a, gsem_a)
            w_wait(buf_b, wsem_b)
            g_start(c + 3, buf_b, gsem_b)
            return carry

        lax.fori_loop(0, nbc // 2 - 1, step, 0)

        c = nbc - 2
        g_wait(buf_a, gsem_a)
        w_start(c, buf_a, wsem_a)
        g_wait(buf_b, gsem_b)
        w_start(c + 1, buf_b, wsem_b)
        w_wait(buf_a, wsem_a)
        w_wait(buf_b, wsem_b)

    return gather


def kernel(input_tensor, indices):
    n_idx = indices.size
    d = input_tensor.shape[-1]
    idx = indices.astype(jnp.int32).reshape(-1)
    wide = _tc_widen(input_tensor.T)
    out2 = _make_gather(n_idx, 2 * d)(wide, idx)
    return out2[:, :d].reshape(indices.shape + (d,))


# speedup vs baseline: 3.0982x; 1.0068x over previous
import functools

import jax
import jax.numpy as jnp
from jax import lax
from jax.experimental import pallas as pl
from jax.experimental.pallas import tpu as pltpu
from jax.experimental.pallas import tpu_sc as plsc

_NC, _NS = 2, 16
_NW = _NC * _NS
_CH = 128
_SPB = 2
_BC = _CH * _SPB
_TBLK = 32768


def _tc_widen(t_t):
    d, v = t_t.shape

    def body(in_ref, out_ref):
        out_ref[:, 0:d] = in_ref[...].T

    return pl.pallas_call(
        body,
        grid=(pl.cdiv(v, _TBLK),),
        in_specs=[pl.BlockSpec((d, _TBLK), lambda i: (0, i))],
        out_specs=pl.BlockSpec((_TBLK, 2 * d), lambda i: (i, 0)),
        out_shape=jax.ShapeDtypeStruct((v, 2 * d), jnp.float32),
        compiler_params=pltpu.CompilerParams(
            vmem_limit_bytes=100 * 1024 * 1024),
    )(t_t)


def _make_gather(n_idx: int, d2: int):
    per_w = n_idx // _NW
    nbc = per_w // _BC
    assert n_idx == _NW * nbc * _BC and nbc >= 5

    mesh = plsc.VectorSubcoreMesh(core_axis_name="c", subcore_axis_name="s")

    @functools.partial(
        pl.kernel,
        out_type=jax.ShapeDtypeStruct((n_idx, d2), jnp.float32),
        mesh=mesh,
        scratch_types=[
            pltpu.VMEM((per_w,), jnp.int32),
            pltpu.VMEM((_BC, d2), jnp.float32),
            pltpu.VMEM((_BC, d2), jnp.float32),
            pltpu.SemaphoreType.DMA,
            pltpu.SemaphoreType.DMA,
            pltpu.SemaphoreType.DMA,
            pltpu.SemaphoreType.DMA,
        ],
        compiler_params=pltpu.CompilerParams(use_tc_tiling_on_sc=True),
    )
    def gather(table_hbm, idx_hbm, out_hbm, idx_v,
               buf_a, buf_b, gsem_a, gsem_b, wsem_a, wsem_b):
        wid = lax.axis_index("s") * _NC + lax.axis_index("c")
        pltpu.sync_copy(idx_hbm.at[pl.ds(wid * per_w, per_w)], idx_v)
        row0 = wid * per_w

        def g_start(c, buf, sem):
            for k in range(_SPB):
                pltpu.make_async_copy(
                    table_hbm.at[idx_v.at[pl.ds((c * _SPB + k) * _CH, _CH)]],
                    buf.at[pl.ds(k * _CH, _CH)], sem).start()

        def g_wait(buf, sem):
            pltpu.make_async_copy(
                table_hbm.at[pl.ds(0, _BC)], buf, sem).wait()

        def w_start(c, buf, sem):
            pltpu.make_async_copy(
                buf, out_hbm.at[pl.ds(row0 + c * _BC, _BC)], sem).start()

        def w_wait(buf, sem):
            pltpu.make_async_copy(
                buf, out_hbm.at[pl.ds(row0, _BC)], sem).wait()

        g_start(0, buf_a, gsem_a)
        g_start(1, buf_b, gsem_b)

        def step(i, carry):
            c = 2 * i
            g_wait(buf_a, gsem_a)
            w_start(c, buf_a, wsem_a)
            g_wait(buf_b, gsem_b)
            w_start(c + 1, buf_b, wsem_b)
            w_wait(buf_a, wsem_a)
            g_start(c + 2, buf_a, gsem_a)
            w_wait(buf_b, wsem_b)
            g_start(c + 3, buf_b, gsem_b)
            return carry

        if nbc % 2 == 0:
            lax.fori_loop(0, nbc // 2 - 1, step, 0)
            c = nbc - 2
            g_wait(buf_a, gsem_a)
            w_start(c, buf_a, wsem_a)
            g_wait(buf_b, gsem_b)
            w_start(c + 1, buf_b, wsem_b)
            w_wait(buf_a, wsem_a)
            w_wait(buf_b, wsem_b)
        else:
            lax.fori_loop(0, (nbc - 3) // 2, step, 0)
            c = nbc - 3
            g_wait(buf_a, gsem_a)
            w_start(c, buf_a, wsem_a)
            g_wait(buf_b, gsem_b)
            w_start(c + 1, buf_b, wsem_b)
            w_wait(buf_a, wsem_a)
            g_start(c + 2, buf_a, gsem_a)
            w_wait(buf_b, wsem_b)
            g_wait(buf_a, gsem_a)
            w_start(c + 2, buf_a, wsem_a)
            w_wait(buf_a, wsem_a)

    return gather


def kernel(input_tensor, indices):
    n_idx = indices.size
    d = input_tensor.shape[-1]
    idx = indices.astype(jnp.int32).reshape(-1)
    wide = _tc_widen(input_tensor.T)
    out2 = _make_gather(n_idx, 2 * d)(wide, idx)
    return out2[:, :d].reshape(indices.shape + (d,))
